# Initial kernel scaffold; baseline (speedup 1.0000x reference)
import jax
import jax.numpy as jnp
from jax.experimental import pallas as pl


def kernel(query_features, prompts, prompt_keys, top_k):
    raise NotImplementedError("write your pallas kernel here")


# speedup vs baseline: 1.0398x; 1.0398x over previous
import functools

import jax
import jax.numpy as jnp
from jax import lax
from jax.experimental import pallas as pl
from jax.experimental.pallas import tpu as pltpu
from jax.experimental.pallas import tpu_sc as plsc

_TOP_K = 4
_NC = 2
_NS = 16
_NW = _NC * _NS


def _topk_body(q_ref, keys_any, idx_ref, knorm_v, sem):
    num_prompts = knorm_v.shape[0]

    @pl.when(pl.program_id(0) == 0)
    def _init():
        copy = pltpu.make_async_copy(keys_any, knorm_v, sem)
        copy.start()
        copy.wait()
        chunk = 1024
        for c in range(num_prompts // chunk):
            sl = pl.ds(c * chunk, chunk)
            blk = knorm_v[sl, :]
            nrm = jnp.sqrt(jnp.sum(blk * blk, axis=1, keepdims=True))
            knorm_v[sl, :] = blk / jnp.maximum(nrm, 1e-12)

    q = q_ref[...]
    qnrm = jnp.sqrt(jnp.sum(q * q, axis=1, keepdims=True))
    q = q / jnp.maximum(qnrm, 1e-12)
    kn = knorm_v[...]
    s = lax.dot_general(q, kn, (((1,), (1,)), ((), ())),
                        preferred_element_type=jnp.float32)
    iota = lax.broadcasted_iota(jnp.int32, s.shape, 1)
    cols = []
    for _ in range(_TOP_K):
        m = jnp.max(s, axis=1, keepdims=True)
        idx_t = jnp.min(jnp.where(s == m, iota, num_prompts), axis=1,
                        keepdims=True)
        cols.append(idx_t)
        s = jnp.where(iota == idx_t, -jnp.inf, s)
    idx_ref[...] = jnp.concatenate(cols, axis=1)


def _topk_tc(query_features, prompt_keys, interpret=False):
    B, D = query_features.shape
    NP = prompt_keys.shape[0]
    QB = 128
    return pl.pallas_call(
        _topk_body,
        grid=(B // QB,),
        in_specs=[
            pl.BlockSpec((QB, D), lambda i: (i, 0)),
            pl.BlockSpec(memory_space=pltpu.MemorySpace.HBM),
        ],
        out_specs=pl.BlockSpec((QB, _TOP_K), lambda i: (i, 0)),
        out_shape=jax.ShapeDtypeStruct((B, _TOP_K), jnp.int32),
        scratch_shapes=[
            pltpu.VMEM((NP, D), jnp.float32),
            pltpu.SemaphoreType.DMA,
        ],
        compiler_params=pltpu.CompilerParams(
            dimension_semantics=("arbitrary",),
        ),
        interpret=interpret,
    )(query_features, prompt_keys)


def _gather_sc(table, idx3, rows_per_worker, chunk):
    NP, RL = table.shape
    nch = rows_per_worker // chunk
    total_rows = _NW * rows_per_worker
    mesh = plsc.VectorSubcoreMesh(core_axis_name="c", subcore_axis_name="s")

    @functools.partial(
        pl.kernel,
        mesh=mesh,
        out_type=jax.ShapeDtypeStruct((total_rows, RL), jnp.float32),
        scratch_types=[
            pltpu.VMEM((nch, chunk), jnp.int32),
            pltpu.VMEM((chunk, RL), jnp.float32),
            pltpu.VMEM((chunk, RL), jnp.float32),
            pltpu.SemaphoreType.DMA,
            pltpu.SemaphoreType.DMA,
            pltpu.SemaphoreType.DMA,
            pltpu.SemaphoreType.DMA,
        ],
    )
    def k(table_hbm, idx_hbm, out_hbm, idx_v, bufa, bufb, sga, sgb, swa, swb):
        wid = lax.axis_index("s") * _NC + lax.axis_index("c")
        base = wid * rows_per_worker
        pltpu.sync_copy(idx_hbm.at[wid], idx_v)

        def g_copy(ch, buf, sem):
            return pltpu.make_async_copy(table_hbm.at[idx_v.at[ch]], buf, sem)

        def w_copy(ch, buf, sem):
            return pltpu.make_async_copy(
                buf, out_hbm.at[pl.ds(base + ch * chunk, chunk)], sem)

        g_copy(0, bufa, sga).start()

        def body(j, carry):
            ch0 = 2 * j
            ch1 = ch0 + 1
            g_copy(ch0, bufa, sga).wait()
            w_copy(ch0, bufa, swa).start()

            @pl.when(j > 0)
            def _():
                w_copy(ch0 - 1, bufb, swb).wait()

            g_copy(ch1, bufb, sgb).start()
            g_copy(ch1, bufb, sgb).wait()
            w_copy(ch1, bufb, swb).start()
            w_copy(ch0, bufa, swa).wait()

            @pl.when(j < nch // 2 - 1)
            def _():
                g_copy(ch0 + 2, bufa, sga).start()

            return carry

        lax.fori_loop(0, nch // 2, body, 0)
        w_copy(nch - 1, bufb, swb).wait()

    return k(table, idx3)


def kernel(query_features, prompts, prompt_keys, top_k):
    B, D = query_features.shape
    NP, PLen, _ = prompts.shape
    idx = _topk_tc(query_features, prompt_keys)
    total_rows = B * _TOP_K
    rows_per_worker = total_rows // _NW
    chunk = 4
    idx3 = idx.reshape(_NW, rows_per_worker // chunk, chunk)
    table = prompts.reshape(NP, PLen * D)
    out = _gather_sc(table, idx3, rows_per_worker, chunk)
    return out.reshape(B, _TOP_K * PLen, D)


# speedup vs baseline: 2.8371x; 2.7286x over previous
import functools

import jax
import jax.numpy as jnp
from jax import lax
from jax.experimental import pallas as pl
from jax.experimental.pallas import tpu as pltpu
from jax.experimental.pallas import tpu_sc as plsc

_TOP_K = 4
_NC = 2
_NS = 16
_NW = _NC * _NS


def _topk_body(q_ref, keys_any, idx_ref, knorm_v, sem):
    num_prompts = knorm_v.shape[0]

    @pl.when(pl.program_id(0) == 0)
    def _init():
        copy = pltpu.make_async_copy(keys_any, knorm_v, sem)
        copy.start()
        copy.wait()
        chunk = 1024
        for c in range(num_prompts // chunk):
            sl = pl.ds(c * chunk, chunk)
            blk = knorm_v[sl, :]
            nrm = jnp.sqrt(jnp.sum(blk * blk, axis=1, keepdims=True))
            knorm_v[sl, :] = blk / jnp.maximum(nrm, 1e-12)

    q = q_ref[...]
    qnrm = jnp.sqrt(jnp.sum(q * q, axis=1, keepdims=True))
    q = q / jnp.maximum(qnrm, 1e-12)
    kn = knorm_v[...]
    s = lax.dot_general(q, kn, (((1,), (1,)), ((), ())),
                        preferred_element_type=jnp.float32)
    iota = lax.broadcasted_iota(jnp.int32, s.shape, 1)
    cols = []
    for _ in range(_TOP_K):
        m = jnp.max(s, axis=1, keepdims=True)
        idx_t = jnp.min(jnp.where(s == m, iota, num_prompts), axis=1,
                        keepdims=True)
        cols.append(idx_t)
        s = jnp.where(iota == idx_t, -jnp.inf, s)
    idx_ref[...] = jnp.concatenate(cols, axis=1)


def _topk_tc(query_features, prompt_keys, interpret=False):
    B, D = query_features.shape
    NP = prompt_keys.shape[0]
    QB = 128
    return pl.pallas_call(
        _topk_body,
        grid=(B // QB,),
        in_specs=[
            pl.BlockSpec((QB, D), lambda i: (i, 0)),
            pl.BlockSpec(memory_space=pltpu.MemorySpace.HBM),
        ],
        out_specs=pl.BlockSpec((QB, _TOP_K), lambda i: (i, 0)),
        out_shape=jax.ShapeDtypeStruct((B, _TOP_K), jnp.int32),
        scratch_shapes=[
            pltpu.VMEM((NP, D), jnp.float32),
            pltpu.SemaphoreType.DMA,
        ],
        compiler_params=pltpu.CompilerParams(
            dimension_semantics=("arbitrary",),
        ),
        interpret=interpret,
    )(query_features, prompt_keys)


def _gather_sc(prompts, idx):
    NP, PLen, D = prompts.shape
    B, K = idx.shape
    qpw = B // _NW
    mesh = plsc.VectorSubcoreMesh(core_axis_name="c", subcore_axis_name="s")

    @functools.partial(
        pl.kernel,
        mesh=mesh,
        out_type=jax.ShapeDtypeStruct((B, K * PLen, D), jnp.float32),
        scratch_types=[
            pltpu.VMEM((qpw, K), jnp.int32),
            pltpu.VMEM((K, PLen, D), jnp.float32),
            pltpu.VMEM((K, PLen, D), jnp.float32),
            pltpu.SemaphoreType.DMA,
            pltpu.SemaphoreType.DMA,
            pltpu.SemaphoreType.DMA,
            pltpu.SemaphoreType.DMA,
        ],
    )
    def k(prompts_hbm, idx_hbm, out_hbm, idx_v, bufa, bufb, sga, sgb, swa, swb):
        wid = lax.axis_index("s") * _NC + lax.axis_index("c")
        qbase = wid * qpw
        pltpu.sync_copy(idx_hbm.at[pl.ds(qbase, qpw)], idx_v)

        def g_copy(q, buf, sem):
            return pltpu.make_async_copy(prompts_hbm.at[idx_v.at[q]], buf, sem)

        def w_copies(q, buf, sem):
            return [
                pltpu.make_async_copy(
                    buf.at[t], out_hbm.at[qbase + q, pl.ds(t * PLen, PLen)],
                    sem)
                for t in range(K)
            ]

        g_copy(0, bufa, sga).start()

        def body(j, carry):
            q0 = 2 * j
            q1 = q0 + 1
            g_copy(q0, bufa, sga).wait()
            for c in w_copies(q0, bufa, swa):
                c.start()

            @pl.when(j > 0)
            def _():
                for c in w_copies(q0 - 1, bufb, swb):
                    c.wait()

            g_copy(q1, bufb, sgb).start()
            g_copy(q1, bufb, sgb).wait()
            for c in w_copies(q1, bufb, swb):
                c.start()
            for c in w_copies(q0, bufa, swa):
                c.wait()

            @pl.when(j < qpw // 2 - 1)
            def _():
                g_copy(q0 + 2, bufa, sga).start()

            return carry

        lax.fori_loop(0, qpw // 2, body, 0)
        for c in w_copies(qpw - 1, bufb, swb):
            c.wait()

    return k(prompts, idx)


def kernel(query_features, prompts, prompt_keys, top_k):
    idx = _topk_tc(query_features, prompt_keys)
    return _gather_sc(prompts, idx)


# speedup vs baseline: 2.8537x; 1.0059x over previous
import functools

import jax
import jax.numpy as jnp
from jax import lax
from jax.experimental import pallas as pl
from jax.experimental.pallas import tpu as pltpu
from jax.experimental.pallas import tpu_sc as plsc

_TOP_K = 4
_NC = 2
_NS = 16
_NW = _NC * _NS


def _topk_body(q_ref, keys_any, idx_ref, knorm_v, sem):
    num_prompts = knorm_v.shape[0]

    @pl.when(pl.program_id(0) == 0)
    def _init():
        copy = pltpu.make_async_copy(keys_any, knorm_v, sem)
        copy.start()
        copy.wait()
        chunk = 1024
        for c in range(num_prompts // chunk):
            sl = pl.ds(c * chunk, chunk)
            blk = knorm_v[sl, :]
            nrm = jnp.sqrt(jnp.sum(blk * blk, axis=1, keepdims=True))
            knorm_v[sl, :] = blk / jnp.maximum(nrm, 1e-12)

    q = q_ref[...]
    qnrm = jnp.sqrt(jnp.sum(q * q, axis=1, keepdims=True))
    q = q / jnp.maximum(qnrm, 1e-12)
    kn = knorm_v[...]
    s = lax.dot_general(q, kn, (((1,), (1,)), ((), ())),
                        preferred_element_type=jnp.float32)
    iota = lax.broadcasted_iota(jnp.int32, s.shape, 1)
    cols = []
    for _ in range(_TOP_K):
        m = jnp.max(s, axis=1, keepdims=True)
        idx_t = jnp.min(jnp.where(s == m, iota, num_prompts), axis=1,
                        keepdims=True)
        cols.append(idx_t)
        s = jnp.where(iota == idx_t, -jnp.inf, s)
    idx_ref[...] = jnp.concatenate(cols, axis=1)


def _topk_tc(query_features, prompt_keys, interpret=False):
    B, D = query_features.shape
    NP = prompt_keys.shape[0]
    QB = 128
    return pl.pallas_call(
        _topk_body,
        grid=(B // QB,),
        in_specs=[
            pl.BlockSpec((QB, D), lambda i: (i, 0)),
            pl.BlockSpec(memory_space=pltpu.MemorySpace.HBM),
        ],
        out_specs=pl.BlockSpec((QB, _TOP_K), lambda i: (i, 0)),
        out_shape=jax.ShapeDtypeStruct((B, _TOP_K), jnp.int32),
        scratch_shapes=[
            pltpu.VMEM((NP, D), jnp.float32),
            pltpu.SemaphoreType.DMA,
        ],
        compiler_params=pltpu.CompilerParams(
            dimension_semantics=("arbitrary",),
        ),
        interpret=interpret,
    )(query_features, prompt_keys)


def _gather_sc(prompts, idx):
    NP, PLen, D = prompts.shape
    B, K = idx.shape
    qpw = B // _NW
    half = K // 2
    nch = qpw * 2
    nbuf = 4
    mesh = plsc.VectorSubcoreMesh(core_axis_name="c", subcore_axis_name="s")

    @functools.partial(
        pl.kernel,
        mesh=mesh,
        out_type=jax.ShapeDtypeStruct((B, K * PLen, D), jnp.float32),
        scratch_types=[
            pltpu.VMEM((qpw, K), jnp.int32),
            [pltpu.VMEM((half, PLen, D), jnp.float32)] * nbuf,
            [pltpu.SemaphoreType.DMA] * nbuf,
            [pltpu.SemaphoreType.DMA] * nbuf,
        ],
    )
    def k(prompts_hbm, idx_hbm, out_hbm, idx_v, bufs, gsems, wsems):
        wid = lax.axis_index("s") * _NC + lax.axis_index("c")
        qbase = wid * qpw
        pltpu.sync_copy(idx_hbm.at[pl.ds(qbase, qpw)], idx_v)

        def g_copy(ch, i):
            return pltpu.make_async_copy(
                prompts_hbm.at[idx_v.at[ch // 2, pl.ds((ch % 2) * half, half)]],
                bufs[i], gsems[i])

        def w_copies(ch, i):
            return [
                pltpu.make_async_copy(
                    bufs[i].at[t],
                    out_hbm.at[qbase + ch // 2,
                               pl.ds(((ch % 2) * half + t) * PLen, PLen)],
                    wsems[i])
                for t in range(half)
            ]

        for c in range(nbuf - 1):
            g_copy(c, c).start()

        def body(j, carry):
            for i in range(nbuf):
                ch = nbuf * j + i
                g_copy(ch, i).wait()
                for c in w_copies(ch, i):
                    c.start()
                prev = ch - 1
                if i == 0:
                    @pl.when(j > 0)
                    def _():
                        for c in w_copies(prev, (nbuf - 1)):
                            c.wait()
                else:
                    for c in w_copies(prev, i - 1):
                        c.wait()
                nxt = ch + nbuf - 1
                if i == 0:
                    g_copy(nxt, nbuf - 1).start()
                else:
                    @pl.when(j < nch // nbuf - 1)
                    def _():
                        g_copy(nxt, i - 1).start()
            return carry

        lax.fori_loop(0, nch // nbuf, body, 0)
        for c in w_copies(nch - 1, nbuf - 1):
            c.wait()

    return k(prompts, idx)


def kernel(query_features, prompts, prompt_keys, top_k):
    idx = _topk_tc(query_features, prompt_keys)
    return _gather_sc(prompts, idx)


# speedup vs baseline: 3.1308x; 1.0971x over previous
import functools

import jax
import jax.numpy as jnp
from jax import lax
from jax.experimental import pallas as pl
from jax.experimental.pallas import tpu as pltpu
from jax.experimental.pallas import tpu_sc as plsc

_TOP_K = 4
_NC = 2
_NS = 16
_NW = _NC * _NS


def _topk_body(q_ref, keys_any, idx_ref, knorm_v, sem):
    num_prompts = knorm_v.shape[0]

    @pl.when(pl.program_id(0) == 0)
    def _init():
        copy = pltpu.make_async_copy(keys_any, knorm_v, sem)
        copy.start()
        copy.wait()
        chunk = 1024
        for c in range(num_prompts // chunk):
            sl = pl.ds(c * chunk, chunk)
            blk = knorm_v[sl, :]
            nrm = jnp.sqrt(jnp.sum(blk * blk, axis=1, keepdims=True))
            knorm_v[sl, :] = blk / jnp.maximum(nrm, 1e-12)

    q = q_ref[...]
    qnrm = jnp.sqrt(jnp.sum(q * q, axis=1, keepdims=True))
    q = q / jnp.maximum(qnrm, 1e-12)
    kn = knorm_v[...]
    s = lax.dot_general(q, kn, (((1,), (1,)), ((), ())),
                        preferred_element_type=jnp.float32)
    iota = lax.broadcasted_iota(jnp.int32, s.shape, 1)
    cols = []
    for _ in range(_TOP_K):
        m = jnp.max(s, axis=1, keepdims=True)
        idx_t = jnp.min(jnp.where(s == m, iota, num_prompts), axis=1,
                        keepdims=True)
        cols.append(idx_t)
        s = jnp.where(iota == idx_t, -jnp.inf, s)
    idx_ref[...] = jnp.concatenate(cols, axis=1)


def _topk_tc(query_features, prompt_keys, interpret=False):
    B, D = query_features.shape
    NP = prompt_keys.shape[0]
    QB = 128
    return pl.pallas_call(
        _topk_body,
        grid=(B // QB,),
        in_specs=[
            pl.BlockSpec((QB, D), lambda i: (i, 0)),
            pl.BlockSpec(memory_space=pltpu.MemorySpace.HBM),
        ],
        out_specs=pl.BlockSpec((QB, _TOP_K), lambda i: (i, 0)),
        out_shape=jax.ShapeDtypeStruct((B, _TOP_K), jnp.int32),
        scratch_shapes=[
            pltpu.VMEM((NP, D), jnp.float32),
            pltpu.SemaphoreType.DMA,
        ],
        compiler_params=pltpu.CompilerParams(
            dimension_semantics=("arbitrary",),
        ),
        interpret=interpret,
    )(query_features, prompt_keys)


def _gather_sc(prompts, idx, out_ref, qoff):
    NP, PLen, D = prompts.shape
    BH, K = idx.shape
    qpw = BH // _NW
    half = K // 2
    nch = qpw * 2
    nbuf = 4
    mesh = plsc.VectorSubcoreMesh(core_axis_name="c", subcore_axis_name="s")

    @functools.partial(
        pl.kernel,
        mesh=mesh,
        out_type=(),
        scratch_types=[
            pltpu.VMEM((qpw, K), jnp.int32),
            [pltpu.VMEM((half, PLen, D), jnp.float32)] * nbuf,
            [pltpu.SemaphoreType.DMA] * nbuf,
            [pltpu.SemaphoreType.DMA] * nbuf,
        ],
    )
    def k(prompts_hbm, idx_hbm, out_hbm, idx_v, bufs, gsems, wsems):
        wid = lax.axis_index("s") * _NC + lax.axis_index("c")
        qbase = qoff + wid * qpw
        pltpu.sync_copy(idx_hbm.at[pl.ds(wid * qpw, qpw)], idx_v)

        def g_copy(ch, i):
            return pltpu.make_async_copy(
                prompts_hbm.at[idx_v.at[ch // 2, pl.ds((ch % 2) * half, half)]],
                bufs[i], gsems[i])

        def w_copies(ch, i):
            return [
                pltpu.make_async_copy(
                    bufs[i].at[t],
                    out_hbm.at[qbase + ch // 2,
                               pl.ds(((ch % 2) * half + t) * PLen, PLen)],
                    wsems[i])
                for t in range(half)
            ]

        for c in range(nbuf - 1):
            g_copy(c, c).start()

        def body(j, carry):
            for i in range(nbuf):
                ch = nbuf * j + i
                g_copy(ch, i).wait()
                for c in w_copies(ch, i):
                    c.start()
                prev = ch - 1
                if i == 0:
                    @pl.when(j > 0)
                    def _():
                        for c in w_copies(prev, (nbuf - 1)):
                            c.wait()
                else:
                    for c in w_copies(prev, i - 1):
                        c.wait()
                nxt = ch + nbuf - 1
                if i == 0:
                    g_copy(nxt, nbuf - 1).start()
                else:
                    @pl.when(j < nch // nbuf - 1)
                    def _():
                        g_copy(nxt, i - 1).start()
            return carry

        lax.fori_loop(0, nch // nbuf, body, 0)
        for c in w_copies(nch - 1, nbuf - 1):
            c.wait()

    k(prompts, idx, out_ref)


def kernel(query_features, prompts, prompt_keys, top_k):
    B, D = query_features.shape
    NP, PLen, _ = prompts.shape
    H = B // 2
    out_ref = jax.empty_ref(
        jax.ShapeDtypeStruct((B, _TOP_K * PLen, D), jnp.float32))
    idx0 = _topk_tc(query_features[:H], prompt_keys)
    _gather_sc(prompts, idx0, out_ref, 0)
    idx1 = _topk_tc(query_features[H:], prompt_keys)
    _gather_sc(prompts, idx1, out_ref, H)
    return out_ref[...]


# speedup vs baseline: 3.1384x; 1.0024x over previous
import functools

import jax
import jax.numpy as jnp
from jax import lax
from jax.experimental import pallas as pl
from jax.experimental.pallas import tpu as pltpu
from jax.experimental.pallas import tpu_sc as plsc

_TOP_K = 4
_NC = 2
_NS = 16
_NW = _NC * _NS


def _topk_body(q_ref, keys_any, idx_ref, knorm_v, sem):
    num_prompts = knorm_v.shape[0]

    @pl.when(pl.program_id(0) == 0)
    def _init():
        copy = pltpu.make_async_copy(keys_any, knorm_v, sem)
        copy.start()
        copy.wait()
        chunk = 1024
        for c in range(num_prompts // chunk):
            sl = pl.ds(c * chunk, chunk)
            blk = knorm_v[sl, :]
            nrm = jnp.sqrt(jnp.sum(blk * blk, axis=1, keepdims=True))
            knorm_v[sl, :] = blk / jnp.maximum(nrm, 1e-12)

    q = q_ref[...]
    qnrm = jnp.sqrt(jnp.sum(q * q, axis=1, keepdims=True))
    q = q / jnp.maximum(qnrm, 1e-12)
    kn = knorm_v[...]
    s = lax.dot_general(q, kn, (((1,), (1,)), ((), ())),
                        preferred_element_type=jnp.float32)
    iota = lax.broadcasted_iota(jnp.int32, s.shape, 1)
    cols = []
    for _ in range(_TOP_K):
        m = jnp.max(s, axis=1, keepdims=True)
        idx_t = jnp.min(jnp.where(s == m, iota, num_prompts), axis=1,
                        keepdims=True)
        cols.append(idx_t)
        s = jnp.where(iota == idx_t, -jnp.inf, s)
    idx_ref[...] = jnp.concatenate(cols, axis=1)


def _topk_tc(query_features, prompt_keys, interpret=False):
    B, D = query_features.shape
    NP = prompt_keys.shape[0]
    QB = 128
    return pl.pallas_call(
        _topk_body,
        grid=(B // QB,),
        in_specs=[
            pl.BlockSpec((QB, D), lambda i: (i, 0)),
            pl.BlockSpec(memory_space=pltpu.MemorySpace.HBM),
        ],
        out_specs=pl.BlockSpec((QB, _TOP_K), lambda i: (i, 0)),
        out_shape=jax.ShapeDtypeStruct((B, _TOP_K), jnp.int32),
        scratch_shapes=[
            pltpu.VMEM((NP, D), jnp.float32),
            pltpu.SemaphoreType.DMA,
        ],
        compiler_params=pltpu.CompilerParams(
            dimension_semantics=("arbitrary",),
        ),
        interpret=interpret,
    )(query_features, prompt_keys)


def _gather_sc(prompts, idx, out_ref, qoff):
    NP, PLen, D = prompts.shape
    BH, K = idx.shape
    qpw = BH // _NW
    half = K // 2
    nch = qpw * 2
    nbuf = 4
    mesh = plsc.VectorSubcoreMesh(core_axis_name="c", subcore_axis_name="s")

    @functools.partial(
        pl.kernel,
        mesh=mesh,
        out_type=(),
        scratch_types=[
            pltpu.VMEM((qpw, K), jnp.int32),
            [pltpu.VMEM((half, PLen, D), jnp.float32)] * nbuf,
            [pltpu.SemaphoreType.DMA] * nbuf,
            [pltpu.SemaphoreType.DMA] * nbuf,
        ],
    )
    def k(prompts_hbm, idx_hbm, out_hbm, idx_v, bufs, gsems, wsems):
        wid = lax.axis_index("s") * _NC + lax.axis_index("c")
        qbase = qoff + wid * qpw
        pltpu.sync_copy(idx_hbm.at[pl.ds(wid * qpw, qpw)], idx_v)

        def g_copy(ch, i):
            return pltpu.make_async_copy(
                prompts_hbm.at[idx_v.at[ch // 2, pl.ds((ch % 2) * half, half)]],
                bufs[i], gsems[i])

        def w_copies(ch, i):
            return [
                pltpu.make_async_copy(
                    bufs[i],
                    out_hbm.at[qbase + ch // 2, pl.ds((ch % 2) * half, half)],
                    wsems[i])
            ]

        for c in range(nbuf - 1):
            g_copy(c, c).start()

        def body(j, carry):
            for i in range(nbuf):
                ch = nbuf * j + i
                g_copy(ch, i).wait()
                for c in w_copies(ch, i):
                    c.start()
                prev = ch - 1
                if i == 0:
                    @pl.when(j > 0)
                    def _():
                        for c in w_copies(prev, (nbuf - 1)):
                            c.wait()
                else:
                    for c in w_copies(prev, i - 1):
                        c.wait()
                nxt = ch + nbuf - 1
                if i == 0:
                    g_copy(nxt, nbuf - 1).start()
                else:
                    @pl.when(j < nch // nbuf - 1)
                    def _():
                        g_copy(nxt, i - 1).start()
            return carry

        lax.fori_loop(0, nch // nbuf, body, 0)
        for c in w_copies(nch - 1, nbuf - 1):
            c.wait()

    k(prompts, idx, out_ref)


def kernel(query_features, prompts, prompt_keys, top_k):
    B, D = query_features.shape
    NP, PLen, _ = prompts.shape
    H = B // 2
    out_ref = jax.empty_ref(
        jax.ShapeDtypeStruct((B, _TOP_K, PLen, D), jnp.float32))
    idx0 = _topk_tc(query_features[:H], prompt_keys)
    _gather_sc(prompts, idx0, out_ref, 0)
    idx1 = _topk_tc(query_features[H:], prompt_keys)
    _gather_sc(prompts, idx1, out_ref, H)
    return out_ref[...].reshape(B, _TOP_K * PLen, D)
